# baseline (speedup 1.0000x reference)
import functools

import jax
import jax.numpy as jnp
from jax import lax
from jax.experimental import pallas as pl
from jax.experimental.pallas import tpu as pltpu
from jax.experimental.pallas import tpu_sc as plsc

DIM = 64
MARGIN = 1.0
C_COEF = 1.0
CHUNK = 128
NC = 2
NS = 16
NW = NC * NS


def _sc_gather(entity_emb, relation_emb, norm_vector_table, eidx, ridx):
    n_ent = eidx.shape[0]
    ipw = n_ent // NW
    cpw = ipw // CHUNK
    n_rel = ridx.shape[0]

    mesh = plsc.VectorSubcoreMesh(core_axis_name="c", subcore_axis_name="s")
    out_type = (
        jax.ShapeDtypeStruct((n_ent, DIM), jnp.float32),
        jax.ShapeDtypeStruct((n_rel, DIM), jnp.float32),
        jax.ShapeDtypeStruct((n_rel, DIM), jnp.float32),
    )

    @functools.partial(
        pl.kernel,
        mesh=mesh,
        out_type=out_type,
        compiler_params=pltpu.CompilerParams(use_tc_tiling_on_sc=False),
        scratch_types=[
            pltpu.VMEM((ipw,), jnp.int32),
            pltpu.VMEM((CHUNK,), jnp.int32),
            pltpu.VMEM((CHUNK, DIM), jnp.float32),
            pltpu.SemaphoreType.DMA,
        ],
    )
    def gather_k(tab, rtab, nvtab, eidx_h, ridx_h, ent_o, r_o, nv_o,
                 idx_v, ridx_v, rows_v, sem):
        wid = lax.axis_index("s") * NC + lax.axis_index("c")
        ibase = wid * ipw
        pltpu.sync_copy(eidx_h.at[pl.ds(ibase, ipw)], idx_v)

        def body(j, carry):
            pltpu.async_copy(
                tab.at[idx_v.at[pl.ds(j * CHUNK, CHUNK)]], rows_v, sem).wait()
            pltpu.sync_copy(rows_v, ent_o.at[pl.ds(ibase + j * CHUNK, CHUNK)])
            return carry

        lax.fori_loop(0, cpw, body, 0)

        pltpu.sync_copy(ridx_h.at[pl.ds(wid * CHUNK, CHUNK)], ridx_v)
        pltpu.async_copy(rtab.at[ridx_v], rows_v, sem).wait()
        pltpu.sync_copy(rows_v, r_o.at[pl.ds(wid * CHUNK, CHUNK)])
        pltpu.async_copy(nvtab.at[ridx_v], rows_v, sem).wait()
        pltpu.sync_copy(rows_v, nv_o.at[pl.ds(wid * CHUNK, CHUNK)])

    return gather_k(entity_emb, relation_emb, norm_vector_table, eidx, ridx)


def _tc_loss(ent_rows, r_rows, nv_rows, B, NEG):
    ngrid = B // CHUNK
    negblk = CHUNK * NEG

    def body(h_r, t_r, nh_r, nt_r, rr_r, nv_r, loss_r,
             acc_m, acc_s, acc_o, acc_r):
        i = pl.program_id(0)

        @pl.when(i == 0)
        def _init():
            acc_m[0, 0] = 0.0
            acc_s[0, 0] = 0.0
            acc_o[0, 0] = 0.0
            acc_r[0, 0] = 0.0

        nv_raw = nv_r[...]
        denom = jnp.maximum(
            jnp.sum(jnp.abs(nv_raw), axis=1, keepdims=True), 1e-12)
        nv = nv_raw / denom
        h = h_r[...]
        t = t_r[...]
        r = rr_r[...]
        d = h - t
        dot = jnp.sum(d * nv, axis=1, keepdims=True)
        e = d - dot * nv + r
        pos = jnp.sum(jnp.abs(e), axis=1, keepdims=True)

        nh = nh_r[...].reshape(CHUNK, NEG, DIM)
        nt = nt_r[...].reshape(CHUNK, NEG, DIM)
        dd = nh - nt
        nvu = nv[:, None, :]
        ndot = jnp.sum(dd * nvu, axis=2, keepdims=True)
        ne = dd - ndot * nvu + r[:, None, :]
        ndist = jnp.sum(jnp.abs(ne), axis=2)

        acc_m[0, 0] += jnp.sum(jnp.maximum(pos + MARGIN - ndist, 0.0))
        acc_s[0, 0] += (
            jnp.sum(jnp.maximum(jnp.sum(h * h, axis=1) - 1.0, 0.0))
            + jnp.sum(jnp.maximum(jnp.sum(t * t, axis=1) - 1.0, 0.0))
            + jnp.sum(jnp.maximum(jnp.sum(nh * nh, axis=2) - 1.0, 0.0))
            + jnp.sum(jnp.maximum(jnp.sum(nt * nt, axis=2) - 1.0, 0.0)))
        acc_o[0, 0] += jnp.sum(jnp.sum(nv * r, axis=1) ** 2)
        acc_r[0, 0] += jnp.sum(jnp.maximum(jnp.sum(r * r, axis=1) - 1.0, 0.0))

        @pl.when(i == ngrid - 1)
        def _fin():
            n_embs = 2.0 * B + 2.0 * B * NEG
            loss_r[0, 0] = (
                acc_m[0, 0] / (B * NEG)
                + C_COEF * (acc_o[0, 0] / B
                            + acc_s[0, 0] / n_embs
                            + acc_r[0, 0] / B))

    out = pl.pallas_call(
        body,
        grid=(ngrid,),
        in_specs=[
            pl.BlockSpec((CHUNK, DIM), lambda i: (i, 0)),
            pl.BlockSpec((CHUNK, DIM), lambda i: (i + ngrid, 0)),
            pl.BlockSpec((negblk, DIM), lambda i: (i + (2 * B) // negblk, 0)),
            pl.BlockSpec((negblk, DIM),
                         lambda i: (i + (2 * B + B * NEG) // negblk, 0)),
            pl.BlockSpec((CHUNK, DIM), lambda i: (i, 0)),
            pl.BlockSpec((CHUNK, DIM), lambda i: (i, 0)),
        ],
        out_specs=pl.BlockSpec(memory_space=pltpu.SMEM),
        out_shape=jax.ShapeDtypeStruct((1, 1), jnp.float32),
        scratch_shapes=[pltpu.SMEM((1, 1), jnp.float32)] * 4,
    )(ent_rows, ent_rows, ent_rows, ent_rows, r_rows, nv_rows)
    return out[0, 0]


def kernel(h, r, t, neg_samples, entity_emb, relation_emb, norm_vector_table):
    B = h.shape[0]
    NEG = neg_samples.shape[1]
    eidx = jnp.concatenate([
        h, t,
        neg_samples[:, :, 0].reshape(-1),
        neg_samples[:, :, 1].reshape(-1),
    ])
    ent_rows, r_rows, nv_rows = _sc_gather(
        entity_emb, relation_emb, norm_vector_table, eidx, r)
    return _tc_loss(ent_rows, r_rows, nv_rows, B, NEG)
